# Initial kernel scaffold; baseline (speedup 1.0000x reference)
import jax
import jax.numpy as jnp
from jax.experimental import pallas as pl


def kernel(triple_idx_i, triple_idx_j, triple_idx_k, idx_i, idx_j, z, positions):
    raise NotImplementedError("write your pallas kernel here")


# speedup vs baseline: 5.7893x; 5.7893x over previous
import jax
import jax.numpy as jnp
import numpy as np
from jax import lax
from jax.experimental import pallas as pl
from jax.experimental.pallas import tpu as pltpu
from jax.experimental.pallas import tpu_sc as plsc

_ORBITALS = '1s 2s 2p 3s 3p 4s 3d 4p 5s 4d 5p 6s 4f 5d 6p 7s 5f 6d 7p 6f 7d 7f'.split()
_POSSIBLE_ELECTRONS = dict(s=2, p=6, d=10, f=14)


def _econf(atomic_num):
    electron_count, last_idx, config = 0, -1, []
    for orb in _ORBITALS:
        if electron_count < atomic_num:
            config.append(_POSSIBLE_ELECTRONS[orb[-1]])
            electron_count += _POSSIBLE_ELECTRONS[orb[-1]]
            last_idx += 1
        else:
            config.append(0)
    if electron_count > atomic_num:
        config[last_idx] -= electron_count - atomic_num
    return config


_ECONF_TABLE = np.array([_econf(i) for i in range(128)], dtype=np.float32)

N_TRIPLES = 800000
ORB = 22
ROW = 3 * ORB + 3
SUB = 128
S = 5
B = S * SUB
NCHUNK = N_TRIPLES // B
GROUPS = B // 16
NC, NS = 2, 16
NW = NC * NS
REC = 16


def _rsqrt_nr(x):
    i = plsc.bitcast(x, jnp.int32)
    y = plsc.bitcast(jnp.int32(0x5F3759DF) - (i >> 1), jnp.float32)
    for _ in range(3):
        y = y * (1.5 - (0.5 * x * y) * y)
    return y


def _sc_body(tji, tjj, tjk, idxj, rec, econf_hbm, out,
             ti_v, tj_v, tk_v, ja_v, ka_v, ri_v, rj_v, rk_v,
             econf_v, outb, sem):
    wid = lax.axis_index("s") * NC + lax.axis_index("c")
    pltpu.sync_copy(econf_hbm, econf_v)
    lane = lax.iota(jnp.int32, 16)

    def chunk_body(it, carry):
        c = wid + it * NW
        base = c * S
        pltpu.sync_copy(tji.at[pl.ds(base, S), :], ti_v)
        pltpu.sync_copy(tjj.at[pl.ds(base, S), :], tj_v)
        pltpu.sync_copy(tjk.at[pl.ds(base, S), :], tk_v)
        for s in range(S):
            pltpu.async_copy(idxj.at[tj_v.at[s]], ja_v.at[s], sem).wait()
            pltpu.async_copy(idxj.at[tk_v.at[s]], ka_v.at[s], sem).wait()
        for s in range(S):
            sl = pl.ds(s * SUB, SUB)
            pltpu.async_copy(rec.at[ti_v.at[s]], ri_v.at[sl], sem).wait()
            pltpu.async_copy(rec.at[ja_v.at[s]], rj_v.at[sl], sem).wait()
            pltpu.async_copy(rec.at[ka_v.at[s]], rk_v.at[sl], sem).wait()

        def group_body(g, gcarry):
            rows = g * 16 + lane

            def col(ref, c_):
                return plsc.load_gather(ref, [rows, jnp.full((16,), c_, jnp.int32)])

            xi, yi, zi = col(ri_v, 0), col(ri_v, 1), col(ri_v, 2)
            xj, yj, zj = col(rj_v, 0), col(rj_v, 1), col(rj_v, 2)
            xk, yk, zk = col(rk_v, 0), col(rk_v, 1), col(rk_v, 2)
            dxj, dyj, dzj = xj - xi, yj - yi, zj - zi
            dxk, dyk, dzk = xk - xi, yk - yi, zk - zi
            d2j = dxj * dxj + dyj * dyj + dzj * dzj
            d2k = dxk * dxk + dyk * dyk + dzk * dzk
            r_ij = d2j * _rsqrt_nr(d2j)
            r_ik = d2k * _rsqrt_nr(d2k)
            dot = dxj * dxk + dyj * dyk + dzj * dzk
            cos = dot / (r_ij * r_ik)

            zni = (lax.convert_element_type(col(ri_v, 3), jnp.int32) & 127) * ORB
            znj = (lax.convert_element_type(col(rj_v, 3), jnp.int32) & 127) * ORB
            znk = (lax.convert_element_type(col(rk_v, 3), jnp.int32) & 127) * ORB

            obase = rows * ROW
            for cc in range(ORB):
                plsc.store_scatter(outb, [obase + cc],
                                   plsc.load_gather(econf_v, [zni + cc]))
                plsc.store_scatter(outb, [obase + (ORB + cc)],
                                   plsc.load_gather(econf_v, [znj + cc]))
                plsc.store_scatter(outb, [obase + (2 * ORB + cc)],
                                   plsc.load_gather(econf_v, [znk + cc]))
            plsc.store_scatter(outb, [obase + (3 * ORB)], r_ij)
            plsc.store_scatter(outb, [obase + (3 * ORB + 1)], r_ik)
            plsc.store_scatter(outb, [obase + (3 * ORB + 2)], cos)
            return gcarry

        lax.fori_loop(0, GROUPS, group_body, 0)
        pltpu.sync_copy(outb, out.at[pl.ds(c * (B * ROW), B * ROW)])
        return carry

    nw_chunks = (NCHUNK - 1 - wid) // NW + 1
    lax.fori_loop(0, nw_chunks, chunk_body, 0)


@jax.jit
def _triple_rep(tji, tjj, tjk, idxj, rec, econf):
    mesh = plsc.VectorSubcoreMesh(core_axis_name="c", subcore_axis_name="s",
                                  num_cores=NC, num_subcores=NS)
    flat = pl.kernel(
        _sc_body,
        out_type=jax.ShapeDtypeStruct((N_TRIPLES * ROW,), jnp.float32),
        mesh=mesh,
        compiler_params=pltpu.CompilerParams(needs_layout_passes=False,
                                             use_tc_tiling_on_sc=False),
        scratch_types=[
            pltpu.VMEM((S, SUB), jnp.int32),
            pltpu.VMEM((S, SUB), jnp.int32),
            pltpu.VMEM((S, SUB), jnp.int32),
            pltpu.VMEM((S, SUB), jnp.int32),
            pltpu.VMEM((S, SUB), jnp.int32),
            pltpu.VMEM((B, REC), jnp.float32),
            pltpu.VMEM((B, REC), jnp.float32),
            pltpu.VMEM((B, REC), jnp.float32),
            pltpu.VMEM((128 * ORB,), jnp.float32),
            pltpu.VMEM((B * ROW,), jnp.float32),
            pltpu.SemaphoreType.DMA,
        ],
    )(tji, tjj, tjk, idxj, rec, econf)
    return flat.reshape(N_TRIPLES, ROW)


def kernel(triple_idx_i, triple_idx_j, triple_idx_k, idx_i, idx_j, z, positions):
    del idx_i
    rec = jnp.concatenate(
        [positions, z.astype(jnp.float32)[:, None],
         jnp.zeros((positions.shape[0], REC - 4), jnp.float32)], axis=1)
    econf = jnp.asarray(_ECONF_TABLE.reshape(-1))
    return _triple_rep(triple_idx_i.reshape(-1, SUB), triple_idx_j.reshape(-1, SUB),
                       triple_idx_k.reshape(-1, SUB), idx_j, rec, econf)


# speedup vs baseline: 7.9332x; 1.3703x over previous
import jax
import jax.numpy as jnp
import numpy as np
from jax import lax
from jax.experimental import pallas as pl
from jax.experimental.pallas import tpu as pltpu
from jax.experimental.pallas import tpu_sc as plsc

_ORBITALS = '1s 2s 2p 3s 3p 4s 3d 4p 5s 4d 5p 6s 4f 5d 6p 7s 5f 6d 7p 6f 7d 7f'.split()
_POSSIBLE_ELECTRONS = dict(s=2, p=6, d=10, f=14)


def _econf(atomic_num):
    electron_count, last_idx, config = 0, -1, []
    for orb in _ORBITALS:
        if electron_count < atomic_num:
            config.append(_POSSIBLE_ELECTRONS[orb[-1]])
            electron_count += _POSSIBLE_ELECTRONS[orb[-1]]
            last_idx += 1
        else:
            config.append(0)
    if electron_count > atomic_num:
        config[last_idx] -= electron_count - atomic_num
    return config


_ECONF_TABLE = np.array([_econf(i) for i in range(128)], dtype=np.float32)

N_TRIPLES = 800000
ORB = 22
ROW = 3 * ORB + 3
SUB = 128
S = 5
B = S * SUB
NCHUNK = N_TRIPLES // B
GROUPS = B // 16
NC, NS = 2, 16
NW = NC * NS
REC = 16


def _rsqrt_nr(x):
    i = plsc.bitcast(x, jnp.int32)
    y = plsc.bitcast(jnp.int32(0x5F3759DF) - (i >> 1), jnp.float32)
    for _ in range(3):
        y = y * (1.5 - (0.5 * x * y) * y)
    return y


def _sc_body(tji, tjj, tjk, idxj, rec, econf_hbm, out,
             ti_v, tj_v, tk_v, ja_v, ka_v, ri_v, rj_v, rk_v,
             econf_v, outb0, outb1, semL, semG1, semG2, semW0, semW1):
    wid = lax.axis_index("s") * NC + lax.axis_index("c")
    pltpu.sync_copy(econf_hbm, econf_v)
    lane = lax.iota(jnp.int32, 16)
    nw_chunks = (NCHUNK - 1 - wid) // NW + 1

    def fire_l(c):
        base = c * S
        pltpu.async_copy(tji.at[pl.ds(base, S), :], ti_v, semL)
        pltpu.async_copy(tjj.at[pl.ds(base, S), :], tj_v, semL)
        pltpu.async_copy(tjk.at[pl.ds(base, S), :], tk_v, semL)

    def wait_l():
        for _ in range(3):
            pltpu.make_async_copy(tji.at[pl.ds(0, S), :], ti_v, semL).wait()

    def fire_g1():
        for s in range(S):
            pltpu.async_copy(idxj.at[tj_v.at[s]], ja_v.at[s], semG1)
            pltpu.async_copy(idxj.at[tk_v.at[s]], ka_v.at[s], semG1)

    def wait_g1():
        for _ in range(2 * S):
            pltpu.make_async_copy(idxj.at[tj_v.at[0]], ja_v.at[0], semG1).wait()

    def fire_g2():
        for s in range(S):
            sl = pl.ds(s * SUB, SUB)
            pltpu.async_copy(rec.at[ti_v.at[s]], ri_v.at[sl], semG2)
            pltpu.async_copy(rec.at[ja_v.at[s]], rj_v.at[sl], semG2)
            pltpu.async_copy(rec.at[ka_v.at[s]], rk_v.at[sl], semG2)

    def wait_g2():
        for _ in range(3 * S):
            pltpu.make_async_copy(rec.at[ti_v.at[0]], ri_v.at[pl.ds(0, SUB)],
                                  semG2).wait()

    def compute(outb):
        def group_body(g, gcarry):
            rows = g * 16 + lane

            def col(ref, c_):
                return plsc.load_gather(ref, [rows, jnp.full((16,), c_, jnp.int32)])

            xi, yi, zi = col(ri_v, 0), col(ri_v, 1), col(ri_v, 2)
            xj, yj, zj = col(rj_v, 0), col(rj_v, 1), col(rj_v, 2)
            xk, yk, zk = col(rk_v, 0), col(rk_v, 1), col(rk_v, 2)
            dxj, dyj, dzj = xj - xi, yj - yi, zj - zi
            dxk, dyk, dzk = xk - xi, yk - yi, zk - zi
            d2j = dxj * dxj + dyj * dyj + dzj * dzj
            d2k = dxk * dxk + dyk * dyk + dzk * dzk
            r_ij = d2j * _rsqrt_nr(d2j)
            r_ik = d2k * _rsqrt_nr(d2k)
            dot = dxj * dxk + dyj * dyk + dzj * dzk
            cos = dot / (r_ij * r_ik)

            zni = (lax.convert_element_type(col(ri_v, 3), jnp.int32) & 127) * ORB
            znj = (lax.convert_element_type(col(rj_v, 3), jnp.int32) & 127) * ORB
            znk = (lax.convert_element_type(col(rk_v, 3), jnp.int32) & 127) * ORB

            obase = rows * ROW
            for cc in range(ORB):
                plsc.store_scatter(outb, [obase + cc],
                                   plsc.load_gather(econf_v, [zni + cc]))
                plsc.store_scatter(outb, [obase + (ORB + cc)],
                                   plsc.load_gather(econf_v, [znj + cc]))
                plsc.store_scatter(outb, [obase + (2 * ORB + cc)],
                                   plsc.load_gather(econf_v, [znk + cc]))
            plsc.store_scatter(outb, [obase + (3 * ORB)], r_ij)
            plsc.store_scatter(outb, [obase + (3 * ORB + 1)], r_ik)
            plsc.store_scatter(outb, [obase + (3 * ORB + 2)], cos)
            return gcarry

        lax.fori_loop(0, GROUPS, group_body, 0)

    def wait_w(outb, semW):
        pltpu.make_async_copy(outb, out.at[pl.ds(0, B * ROW)], semW).wait()

    def one_chunk(it, outb, semW):
        c = wid + it * NW
        wait_l()
        fire_g1()
        wait_g1()
        fire_g2()
        wait_g2()

        @pl.when(it + 1 < nw_chunks)
        def _():
            fire_l(c + NW)

        @pl.when(it >= 2)
        def _():
            wait_w(outb, semW)

        compute(outb)
        pltpu.async_copy(outb, out.at[pl.ds(c * (B * ROW), B * ROW)], semW)

    fire_l(wid)

    def pair_body(it2, carry):
        it0 = it2 * 2

        @pl.when(it0 < nw_chunks)
        def _():
            one_chunk(it0, outb0, semW0)

        @pl.when(it0 + 1 < nw_chunks)
        def _():
            one_chunk(it0 + 1, outb1, semW1)

        return carry

    lax.fori_loop(0, (nw_chunks + 1) // 2, pair_body, 0)

    @pl.when(nw_chunks >= 1)
    def _():
        wait_w(outb0, semW0)

    @pl.when(nw_chunks >= 2)
    def _():
        wait_w(outb1, semW1)


@jax.jit
def _triple_rep(tji, tjj, tjk, idxj, rec, econf):
    mesh = plsc.VectorSubcoreMesh(core_axis_name="c", subcore_axis_name="s",
                                  num_cores=NC, num_subcores=NS)
    flat = pl.kernel(
        _sc_body,
        out_type=jax.ShapeDtypeStruct((N_TRIPLES * ROW,), jnp.float32),
        mesh=mesh,
        compiler_params=pltpu.CompilerParams(needs_layout_passes=False,
                                             use_tc_tiling_on_sc=False),
        scratch_types=[
            pltpu.VMEM((S, SUB), jnp.int32),
            pltpu.VMEM((S, SUB), jnp.int32),
            pltpu.VMEM((S, SUB), jnp.int32),
            pltpu.VMEM((S, SUB), jnp.int32),
            pltpu.VMEM((S, SUB), jnp.int32),
            pltpu.VMEM((B, REC), jnp.float32),
            pltpu.VMEM((B, REC), jnp.float32),
            pltpu.VMEM((B, REC), jnp.float32),
            pltpu.VMEM((128 * ORB,), jnp.float32),
            pltpu.VMEM((B * ROW,), jnp.float32),
            pltpu.VMEM((B * ROW,), jnp.float32),
            pltpu.SemaphoreType.DMA,
            pltpu.SemaphoreType.DMA,
            pltpu.SemaphoreType.DMA,
            pltpu.SemaphoreType.DMA,
            pltpu.SemaphoreType.DMA,
        ],
    )(tji, tjj, tjk, idxj, rec, econf)
    return flat.reshape(N_TRIPLES, ROW)


def kernel(triple_idx_i, triple_idx_j, triple_idx_k, idx_i, idx_j, z, positions):
    del idx_i
    rec = jnp.concatenate(
        [positions, z.astype(jnp.float32)[:, None],
         jnp.zeros((positions.shape[0], REC - 4), jnp.float32)], axis=1)
    econf = jnp.asarray(_ECONF_TABLE.reshape(-1))
    return _triple_rep(triple_idx_i.reshape(-1, SUB), triple_idx_j.reshape(-1, SUB),
                       triple_idx_k.reshape(-1, SUB), idx_j, rec, econf)


# speedup vs baseline: 8.7308x; 1.1005x over previous
import jax
import jax.numpy as jnp
import numpy as np
from jax import lax
from jax.experimental import pallas as pl
from jax.experimental.pallas import tpu as pltpu
from jax.experimental.pallas import tpu_sc as plsc

_ORBITALS = '1s 2s 2p 3s 3p 4s 3d 4p 5s 4d 5p 6s 4f 5d 6p 7s 5f 6d 7p 6f 7d 7f'.split()
_POSSIBLE_ELECTRONS = dict(s=2, p=6, d=10, f=14)


def _econf(atomic_num):
    electron_count, last_idx, config = 0, -1, []
    for orb in _ORBITALS:
        if electron_count < atomic_num:
            config.append(_POSSIBLE_ELECTRONS[orb[-1]])
            electron_count += _POSSIBLE_ELECTRONS[orb[-1]]
            last_idx += 1
        else:
            config.append(0)
    if electron_count > atomic_num:
        config[last_idx] -= electron_count - atomic_num
    return config


_ECONF_TABLE = np.array([_econf(i) for i in range(128)], dtype=np.float32)

N_TRIPLES = 800000
ORB = 22
ROW = 3 * ORB + 3
SUB = 128
S = 5
B = S * SUB
NCHUNK = N_TRIPLES // B
GROUPS = B // 16
NC, NS = 2, 16
NW = NC * NS
REC = 16


def _rsqrt_nr(x):
    i = plsc.bitcast(x, jnp.int32)
    y = plsc.bitcast(jnp.int32(0x5F3759DF) - (i >> 1), jnp.float32)
    for _ in range(3):
        y = y * (1.5 - (0.5 * x * y) * y)
    return y


def _sc_body(tji, tjj, tjk, idxj, rec, econf_hbm, out,
             ti_v, tj_v, tk_v, ja_v, ka_v, ri_v, rj_v, rk_v,
             econf_v, outb0, outb1, semL, semG1, semG2, semW0, semW1):
    wid = lax.axis_index("s") * NC + lax.axis_index("c")
    pltpu.sync_copy(econf_hbm, econf_v)
    lane = lax.iota(jnp.int32, 16)
    nw_chunks = (NCHUNK - 1 - wid) // NW + 1

    def fire_l(c):
        base = c * S
        pltpu.async_copy(tji.at[pl.ds(base, S), :], ti_v, semL)
        pltpu.async_copy(tjj.at[pl.ds(base, S), :], tj_v, semL)
        pltpu.async_copy(tjk.at[pl.ds(base, S), :], tk_v, semL)

    def wait_l():
        for _ in range(3):
            pltpu.make_async_copy(tji.at[pl.ds(0, S), :], ti_v, semL).wait()

    def fire_g1():
        for s in range(S):
            pltpu.async_copy(idxj.at[tj_v.at[s]], ja_v.at[s], semG1)
            pltpu.async_copy(idxj.at[tk_v.at[s]], ka_v.at[s], semG1)

    def wait_g1():
        for _ in range(2 * S):
            pltpu.make_async_copy(idxj.at[tj_v.at[0]], ja_v.at[0], semG1).wait()

    def fire_g2():
        for s in range(S):
            sl = pl.ds(s * SUB, SUB)
            pltpu.async_copy(rec.at[ti_v.at[s]], ri_v.at[sl], semG2)
            pltpu.async_copy(rec.at[ja_v.at[s]], rj_v.at[sl], semG2)
            pltpu.async_copy(rec.at[ka_v.at[s]], rk_v.at[sl], semG2)

    def wait_g2():
        for _ in range(3 * S):
            pltpu.make_async_copy(rec.at[ti_v.at[0]], ri_v.at[pl.ds(0, SUB)],
                                  semG2).wait()

    def compute(outb):
        def group_body(g, gcarry):
            rows = g * 16 + lane

            def col(ref, c_):
                return plsc.load_gather(ref, [rows, jnp.full((16,), c_, jnp.int32)])

            xi, yi, zi = col(ri_v, 0), col(ri_v, 1), col(ri_v, 2)
            xj, yj, zj = col(rj_v, 0), col(rj_v, 1), col(rj_v, 2)
            xk, yk, zk = col(rk_v, 0), col(rk_v, 1), col(rk_v, 2)
            dxj, dyj, dzj = xj - xi, yj - yi, zj - zi
            dxk, dyk, dzk = xk - xi, yk - yi, zk - zi
            d2j = dxj * dxj + dyj * dyj + dzj * dzj
            d2k = dxk * dxk + dyk * dyk + dzk * dzk
            r_ij = d2j * _rsqrt_nr(d2j)
            r_ik = d2k * _rsqrt_nr(d2k)
            dot = dxj * dxk + dyj * dyk + dzj * dzk
            cos = dot / (r_ij * r_ik)

            zni = (lax.convert_element_type(col(ri_v, 3), jnp.int32) & 127) * ORB
            znj = (lax.convert_element_type(col(rj_v, 3), jnp.int32) & 127) * ORB
            znk = (lax.convert_element_type(col(rk_v, 3), jnp.int32) & 127) * ORB

            def cv(c_):
                return jnp.full((16,), c_, jnp.int32)

            for cc in range(ORB):
                plsc.store_scatter(outb, [rows, cv(cc)],
                                   plsc.load_gather(econf_v, [zni + cc]))
                plsc.store_scatter(outb, [rows, cv(ORB + cc)],
                                   plsc.load_gather(econf_v, [znj + cc]))
                plsc.store_scatter(outb, [rows, cv(2 * ORB + cc)],
                                   plsc.load_gather(econf_v, [znk + cc]))
            plsc.store_scatter(outb, [rows, cv(3 * ORB)], r_ij)
            plsc.store_scatter(outb, [rows, cv(3 * ORB + 1)], r_ik)
            plsc.store_scatter(outb, [rows, cv(3 * ORB + 2)], cos)
            return gcarry

        lax.fori_loop(0, GROUPS, group_body, 0)

    def wait_w(outb, semW):
        pltpu.make_async_copy(outb, out.at[pl.ds(0, B), :], semW).wait()

    def one_chunk(it, outb, semW):
        c = wid + it * NW
        wait_l()
        fire_g1()
        wait_g1()
        fire_g2()
        wait_g2()

        @pl.when(it + 1 < nw_chunks)
        def _():
            fire_l(c + NW)

        @pl.when(it >= 2)
        def _():
            wait_w(outb, semW)

        compute(outb)
        pltpu.async_copy(outb, out.at[pl.ds(c * B, B), :], semW)

    fire_l(wid)

    def pair_body(it2, carry):
        it0 = it2 * 2

        @pl.when(it0 < nw_chunks)
        def _():
            one_chunk(it0, outb0, semW0)

        @pl.when(it0 + 1 < nw_chunks)
        def _():
            one_chunk(it0 + 1, outb1, semW1)

        return carry

    lax.fori_loop(0, (nw_chunks + 1) // 2, pair_body, 0)

    @pl.when(nw_chunks >= 1)
    def _():
        wait_w(outb0, semW0)

    @pl.when(nw_chunks >= 2)
    def _():
        wait_w(outb1, semW1)


@jax.jit
def _triple_rep(tji, tjj, tjk, idxj, rec, econf):
    mesh = plsc.VectorSubcoreMesh(core_axis_name="c", subcore_axis_name="s",
                                  num_cores=NC, num_subcores=NS)
    return pl.kernel(
        _sc_body,
        out_type=jax.ShapeDtypeStruct((N_TRIPLES, ROW), jnp.float32),
        mesh=mesh,
        compiler_params=pltpu.CompilerParams(needs_layout_passes=False,
                                             use_tc_tiling_on_sc=False),
        scratch_types=[
            pltpu.VMEM((S, SUB), jnp.int32),
            pltpu.VMEM((S, SUB), jnp.int32),
            pltpu.VMEM((S, SUB), jnp.int32),
            pltpu.VMEM((S, SUB), jnp.int32),
            pltpu.VMEM((S, SUB), jnp.int32),
            pltpu.VMEM((B, REC), jnp.float32),
            pltpu.VMEM((B, REC), jnp.float32),
            pltpu.VMEM((B, REC), jnp.float32),
            pltpu.VMEM((128 * ORB,), jnp.float32),
            pltpu.VMEM((B, ROW), jnp.float32),
            pltpu.VMEM((B, ROW), jnp.float32),
            pltpu.SemaphoreType.DMA,
            pltpu.SemaphoreType.DMA,
            pltpu.SemaphoreType.DMA,
            pltpu.SemaphoreType.DMA,
            pltpu.SemaphoreType.DMA,
        ],
    )(tji, tjj, tjk, idxj, rec, econf)


def kernel(triple_idx_i, triple_idx_j, triple_idx_k, idx_i, idx_j, z, positions):
    del idx_i
    rec = jnp.concatenate(
        [positions, z.astype(jnp.float32)[:, None],
         jnp.zeros((positions.shape[0], REC - 4), jnp.float32)], axis=1)
    econf = jnp.asarray(_ECONF_TABLE.reshape(-1))
    return _triple_rep(triple_idx_i.reshape(-1, SUB), triple_idx_j.reshape(-1, SUB),
                       triple_idx_k.reshape(-1, SUB), idx_j, rec, econf)


# speedup vs baseline: 9.1127x; 1.0437x over previous
import jax
import jax.numpy as jnp
import numpy as np
from jax import lax
from jax.experimental import pallas as pl
from jax.experimental.pallas import tpu as pltpu
from jax.experimental.pallas import tpu_sc as plsc

_ORBITALS = '1s 2s 2p 3s 3p 4s 3d 4p 5s 4d 5p 6s 4f 5d 6p 7s 5f 6d 7p 6f 7d 7f'.split()
_POSSIBLE_ELECTRONS = dict(s=2, p=6, d=10, f=14)


def _econf(atomic_num):
    electron_count, last_idx, config = 0, -1, []
    for orb in _ORBITALS:
        if electron_count < atomic_num:
            config.append(_POSSIBLE_ELECTRONS[orb[-1]])
            electron_count += _POSSIBLE_ELECTRONS[orb[-1]]
            last_idx += 1
        else:
            config.append(0)
    if electron_count > atomic_num:
        config[last_idx] -= electron_count - atomic_num
    return config


_ECONF_TABLE = np.array([_econf(i) for i in range(128)], dtype=np.float32)

N_TRIPLES = 800000
ORB = 22
ROW = 3 * ORB + 3
SUB = 128
S = 2
B = S * SUB
NCHUNK = N_TRIPLES // B
GROUPS = B // 16
HALF = GROUPS // 2
NC, NS = 2, 16
NW = NC * NS
REC = 16


def _rsqrt_nr(x):
    i = plsc.bitcast(x, jnp.int32)
    y = plsc.bitcast(jnp.int32(0x5F3759DF) - (i >> 1), jnp.float32)
    for _ in range(3):
        y = y * (1.5 - (0.5 * x * y) * y)
    return y


def _sc_body(tji, tjj, tjk, idxj, rec, econf_hbm, out,
             ti0, tj0, tk0, ja0, ka0, ri0, rj0, rk0,
             ti1, tj1, tk1, ja1, ka1, ri1, rj1, rk1,
             econf_v, outb0, outb1,
             semL0, semL1, semG10, semG11, semG20, semG21, semW0, semW1):
    wid = lax.axis_index("s") * NC + lax.axis_index("c")
    pltpu.sync_copy(econf_hbm, econf_v)
    lane = lax.iota(jnp.int32, 16)
    nw_chunks = (NCHUNK - 1 - wid) // NW + 1

    sets = [
        dict(ti=ti0, tj=tj0, tk=tk0, ja=ja0, ka=ka0, ri=ri0, rj=rj0, rk=rk0,
             semL=semL0, semG1=semG10, semG2=semG20),
        dict(ti=ti1, tj=tj1, tk=tk1, ja=ja1, ka=ka1, ri=ri1, rj=rj1, rk=rk1,
             semL=semL1, semG1=semG11, semG2=semG21),
    ]
    outbs = [outb0, outb1]
    semWs = [semW0, semW1]

    def fire_l(c, st):
        base = c * S
        pltpu.async_copy(tji.at[pl.ds(base, S), :], st['ti'], st['semL'])
        pltpu.async_copy(tjj.at[pl.ds(base, S), :], st['tj'], st['semL'])
        pltpu.async_copy(tjk.at[pl.ds(base, S), :], st['tk'], st['semL'])

    def wait_l(st):
        for _ in range(3):
            pltpu.make_async_copy(tji.at[pl.ds(0, S), :], st['ti'],
                                  st['semL']).wait()

    def fire_g1(st):
        for s in range(S):
            pltpu.async_copy(idxj.at[st['tj'].at[s]], st['ja'].at[s], st['semG1'])
            pltpu.async_copy(idxj.at[st['tk'].at[s]], st['ka'].at[s], st['semG1'])

    def wait_g1(st):
        for _ in range(2 * S):
            pltpu.make_async_copy(idxj.at[st['tj'].at[0]], st['ja'].at[0],
                                  st['semG1']).wait()

    def fire_g2(st):
        for s in range(S):
            sl = pl.ds(s * SUB, SUB)
            pltpu.async_copy(rec.at[st['ti'].at[s]], st['ri'].at[sl], st['semG2'])
            pltpu.async_copy(rec.at[st['ja'].at[s]], st['rj'].at[sl], st['semG2'])
            pltpu.async_copy(rec.at[st['ka'].at[s]], st['rk'].at[sl], st['semG2'])

    def wait_g2(st):
        for _ in range(3 * S):
            pltpu.make_async_copy(rec.at[st['ti'].at[0]],
                                  st['ri'].at[pl.ds(0, SUB)], st['semG2']).wait()

    def compute(outb, st, glo, ghi):
        ri_v, rj_v, rk_v = st['ri'], st['rj'], st['rk']

        def group_body(g, gcarry):
            rows = g * 16 + lane

            def col(ref, c_):
                return plsc.load_gather(ref, [rows, jnp.full((16,), c_, jnp.int32)])

            xi, yi, zi = col(ri_v, 0), col(ri_v, 1), col(ri_v, 2)
            xj, yj, zj = col(rj_v, 0), col(rj_v, 1), col(rj_v, 2)
            xk, yk, zk = col(rk_v, 0), col(rk_v, 1), col(rk_v, 2)
            dxj, dyj, dzj = xj - xi, yj - yi, zj - zi
            dxk, dyk, dzk = xk - xi, yk - yi, zk - zi
            d2j = dxj * dxj + dyj * dyj + dzj * dzj
            d2k = dxk * dxk + dyk * dyk + dzk * dzk
            r_ij = d2j * _rsqrt_nr(d2j)
            r_ik = d2k * _rsqrt_nr(d2k)
            dot = dxj * dxk + dyj * dyk + dzj * dzk
            cos = dot / (r_ij * r_ik)

            zni = (lax.convert_element_type(col(ri_v, 3), jnp.int32) & 127) * ORB
            znj = (lax.convert_element_type(col(rj_v, 3), jnp.int32) & 127) * ORB
            znk = (lax.convert_element_type(col(rk_v, 3), jnp.int32) & 127) * ORB

            def cv(c_):
                return jnp.full((16,), c_, jnp.int32)

            for cc in range(ORB):
                plsc.store_scatter(outb, [rows, cv(cc)],
                                   plsc.load_gather(econf_v, [zni + cc]))
                plsc.store_scatter(outb, [rows, cv(ORB + cc)],
                                   plsc.load_gather(econf_v, [znj + cc]))
                plsc.store_scatter(outb, [rows, cv(2 * ORB + cc)],
                                   plsc.load_gather(econf_v, [znk + cc]))
            plsc.store_scatter(outb, [rows, cv(3 * ORB)], r_ij)
            plsc.store_scatter(outb, [rows, cv(3 * ORB + 1)], r_ik)
            plsc.store_scatter(outb, [rows, cv(3 * ORB + 2)], cos)
            return gcarry

        lax.fori_loop(glo, ghi, group_body, 0)

    def wait_w(p):
        pltpu.make_async_copy(outbs[p], out.at[pl.ds(0, B), :], semWs[p]).wait()

    def one_chunk(it, p):
        c = wid + it * NW
        st, stq = sets[p], sets[1 - p]
        wait_g2(st)

        @pl.when(it + 2 < nw_chunks)
        def _():
            fire_l(c + 2 * NW, st)

        @pl.when(it >= 2)
        def _():
            wait_w(p)

        compute(outbs[p], st, 0, HALF)

        @pl.when(it + 1 < nw_chunks)
        def _():
            wait_l(stq)
            fire_g1(stq)

        compute(outbs[p], st, HALF, GROUPS)

        @pl.when(it + 1 < nw_chunks)
        def _():
            wait_g1(stq)
            fire_g2(stq)

        pltpu.async_copy(outbs[p], out.at[pl.ds(c * B, B), :], semWs[p])

    fire_l(wid, sets[0])
    wait_l(sets[0])
    fire_g1(sets[0])
    wait_g1(sets[0])
    fire_g2(sets[0])

    @pl.when(nw_chunks > 1)
    def _():
        fire_l(wid + NW, sets[1])

    def pair_body(it2, carry):
        it0 = it2 * 2

        @pl.when(it0 < nw_chunks)
        def _():
            one_chunk(it0, 0)

        @pl.when(it0 + 1 < nw_chunks)
        def _():
            one_chunk(it0 + 1, 1)

        return carry

    lax.fori_loop(0, (nw_chunks + 1) // 2, pair_body, 0)

    @pl.when(nw_chunks >= 1)
    def _():
        wait_w(0)

    @pl.when(nw_chunks >= 2)
    def _():
        wait_w(1)


@jax.jit
def _triple_rep(tji, tjj, tjk, idxj, rec, econf):
    mesh = plsc.VectorSubcoreMesh(core_axis_name="c", subcore_axis_name="s",
                                  num_cores=NC, num_subcores=NS)
    idx_buf = pltpu.VMEM((S, SUB), jnp.int32)
    rec_buf = pltpu.VMEM((B, REC), jnp.float32)
    return pl.kernel(
        _sc_body,
        out_type=jax.ShapeDtypeStruct((N_TRIPLES, ROW), jnp.float32),
        mesh=mesh,
        compiler_params=pltpu.CompilerParams(needs_layout_passes=False,
                                             use_tc_tiling_on_sc=False),
        scratch_types=(
            [idx_buf] * 5 + [rec_buf] * 3 +
            [idx_buf] * 5 + [rec_buf] * 3 +
            [pltpu.VMEM((128 * ORB,), jnp.float32),
             pltpu.VMEM((B, ROW), jnp.float32),
             pltpu.VMEM((B, ROW), jnp.float32)] +
            [pltpu.SemaphoreType.DMA] * 8
        ),
    )(tji, tjj, tjk, idxj, rec, econf)


def kernel(triple_idx_i, triple_idx_j, triple_idx_k, idx_i, idx_j, z, positions):
    del idx_i
    rec = jnp.concatenate(
        [positions, z.astype(jnp.float32)[:, None],
         jnp.zeros((positions.shape[0], REC - 4), jnp.float32)], axis=1)
    econf = jnp.asarray(_ECONF_TABLE.reshape(-1))
    return _triple_rep(triple_idx_i.reshape(-1, SUB), triple_idx_j.reshape(-1, SUB),
                       triple_idx_k.reshape(-1, SUB), idx_j, rec, econf)
